# baseline (speedup 1.0000x reference)
import jax
import jax.numpy as jnp
from jax.experimental import pallas as pl
from jax.experimental.pallas import tpu as pltpu

_MARGIN = 1.0
_NUM_CLASSES = 32
_NBLK = 4


def _dot_t(a, b):
    return jax.lax.dot_general(
        a, b, (((1,), (1,)), ((), ())), preferred_element_type=jnp.float32)


def _triplet_loss_kernel(emb_ref, lab_row_ref, out_ref):
    emb = emb_ref[...]
    n, f = emb.shape
    blk = n // _NBLK
    emb_bf = emb.astype(jnp.bfloat16)
    embm2_bf = emb_bf * jnp.bfloat16(-2.0)
    embsq = emb * emb
    ones_f = jnp.ones((1, f), jnp.float32)
    d_row = _dot_t(ones_f, embsq)
    lab_row = lab_row_ref[...]

    col = jax.lax.broadcasted_iota(jnp.int32, (blk, n), 1)
    row = jax.lax.broadcasted_iota(jnp.int32, (blk, n), 0)
    col_minus_row = col - row
    ones_blk = jnp.ones((1, blk), jnp.float32)
    colsum = jnp.zeros((1, n), jnp.float32)
    for b in range(_NBLK):
        g2 = _dot_t(emb_bf[b * blk:(b + 1) * blk], embm2_bf)
        t = g2 + d_row
        lab_col = jnp.transpose(lab_row[:, b * blk:(b + 1) * blk])
        same = lab_col == lab_row
        tneg = jnp.where(same, jnp.float32(jnp.inf), t)
        mn = jnp.min(tneg, axis=1, keepdims=True)
        cc = _MARGIN - jnp.where(jnp.isinf(mn), t[:, 0:1], mn)
        upper = col_minus_row > b * blk
        losses = jnp.maximum(t + cc, 0.0)
        lm = jnp.where(same & upper, losses, 0.0)
        colsum = colsum + jax.lax.dot_general(
            ones_blk, lm, (((1,), (0,)), ((), ())),
            preferred_element_type=jnp.float32)
    loss_sum = jnp.sum(colsum, keepdims=True)

    lvals = jax.lax.broadcasted_iota(jnp.int32, (_NUM_CLASSES, n), 0)
    onehot = (lab_row == lvals).astype(jnp.float32)
    m = jnp.sum(onehot, axis=1, keepdims=True)
    cnt = 0.5 * (jnp.sum(m * m, keepdims=True) - jnp.float32(n))
    out_ref[...] = (loss_sum / cnt).reshape(1, 1)


def kernel(embeddings, target):
    b = embeddings.shape[0]
    lab = target.astype(jnp.int32)
    out = pl.pallas_call(
        _triplet_loss_kernel,
        out_shape=jax.ShapeDtypeStruct((1, 1), jnp.float32),
    )(embeddings, lab.reshape(1, b))
    return out[0, 0]


# speedup vs baseline: 1.1376x; 1.1376x over previous
import jax
import jax.numpy as jnp
from jax.experimental import pallas as pl
from jax.experimental.pallas import tpu as pltpu

_MARGIN = 1.0
_NUM_CLASSES = 32
_NBLK = 2


def _dot_t(a, b):
    return jax.lax.dot_general(
        a, b, (((1,), (1,)), ((), ())), preferred_element_type=jnp.float32)


def _triplet_loss_kernel(emb_ref, lab_row_ref, out_ref):
    emb = emb_ref[...]
    n, f = emb.shape
    blk = n // _NBLK
    emb_bf = emb.astype(jnp.bfloat16)
    embm2_bf = emb_bf * jnp.bfloat16(-2.0)
    embsq = emb * emb
    ones_f = jnp.ones((1, f), jnp.float32)
    d_row = _dot_t(ones_f, embsq)
    lab_row = lab_row_ref[...]

    col = jax.lax.broadcasted_iota(jnp.int32, (blk, n), 1)
    row = jax.lax.broadcasted_iota(jnp.int32, (blk, n), 0)
    col_minus_row = col - row
    ones_blk = jnp.ones((1, blk), jnp.float32)
    colsum = jnp.zeros((1, n), jnp.float32)
    for b in range(_NBLK):
        g2 = _dot_t(emb_bf[b * blk:(b + 1) * blk], embm2_bf)
        t = g2 + d_row
        lab_col = jnp.transpose(lab_row[:, b * blk:(b + 1) * blk])
        same = lab_col == lab_row
        tneg = jnp.where(same, jnp.float32(jnp.inf), t)
        mn = jnp.min(tneg, axis=1, keepdims=True)
        cc = _MARGIN - jnp.where(jnp.isinf(mn), t[:, 0:1], mn)
        upper = col_minus_row > b * blk
        losses = jnp.maximum(t + cc, 0.0)
        lm = jnp.where(same & upper, losses, 0.0)
        colsum = colsum + jax.lax.dot_general(
            ones_blk, lm, (((1,), (0,)), ((), ())),
            preferred_element_type=jnp.float32)
    loss_sum = jnp.sum(colsum, keepdims=True)

    lvals = jax.lax.broadcasted_iota(jnp.int32, (_NUM_CLASSES, n), 0)
    onehot = (lab_row == lvals).astype(jnp.float32)
    m = jnp.sum(onehot, axis=1, keepdims=True)
    cnt = 0.5 * (jnp.sum(m * m, keepdims=True) - jnp.float32(n))
    out_ref[...] = (loss_sum / cnt).reshape(1, 1)


def kernel(embeddings, target):
    b = embeddings.shape[0]
    lab = target.astype(jnp.int32)
    out = pl.pallas_call(
        _triplet_loss_kernel,
        out_shape=jax.ShapeDtypeStruct((1, 1), jnp.float32),
    )(embeddings, lab.reshape(1, b))
    return out[0, 0]


# speedup vs baseline: 1.1692x; 1.0278x over previous
import jax
import jax.numpy as jnp
from jax.experimental import pallas as pl
from jax.experimental.pallas import tpu as pltpu

_MARGIN = 1.0
_NUM_CLASSES = 32
_NBLK = 1


def _dot_t(a, b):
    return jax.lax.dot_general(
        a, b, (((1,), (1,)), ((), ())), preferred_element_type=jnp.float32)


def _triplet_loss_kernel(emb_ref, lab_row_ref, out_ref):
    emb = emb_ref[...]
    n, f = emb.shape
    blk = n // _NBLK
    emb_bf = emb.astype(jnp.bfloat16)
    embm2_bf = emb_bf * jnp.bfloat16(-2.0)
    embsq = emb * emb
    ones_f = jnp.ones((1, f), jnp.float32)
    d_row = _dot_t(ones_f, embsq)
    lab_row = lab_row_ref[...]

    col = jax.lax.broadcasted_iota(jnp.int32, (blk, n), 1)
    row = jax.lax.broadcasted_iota(jnp.int32, (blk, n), 0)
    col_minus_row = col - row
    ones_blk = jnp.ones((1, blk), jnp.float32)
    colsum = jnp.zeros((1, n), jnp.float32)
    for b in range(_NBLK):
        g2 = _dot_t(emb_bf[b * blk:(b + 1) * blk], embm2_bf)
        t = g2 + d_row
        lab_col = jnp.transpose(lab_row[:, b * blk:(b + 1) * blk])
        same = lab_col == lab_row
        tneg = jnp.where(same, jnp.float32(jnp.inf), t)
        mn = jnp.min(tneg, axis=1, keepdims=True)
        cc = _MARGIN - jnp.where(jnp.isinf(mn), t[:, 0:1], mn)
        upper = col_minus_row > b * blk
        losses = jnp.maximum(t + cc, 0.0)
        lm = jnp.where(same & upper, losses, 0.0)
        colsum = colsum + jax.lax.dot_general(
            ones_blk, lm, (((1,), (0,)), ((), ())),
            preferred_element_type=jnp.float32)
    loss_sum = jnp.sum(colsum, keepdims=True)

    lvals = jax.lax.broadcasted_iota(jnp.int32, (_NUM_CLASSES, n), 0)
    onehot = (lab_row == lvals).astype(jnp.float32)
    m = jnp.sum(onehot, axis=1, keepdims=True)
    cnt = 0.5 * (jnp.sum(m * m, keepdims=True) - jnp.float32(n))
    out_ref[...] = (loss_sum / cnt).reshape(1, 1)


def kernel(embeddings, target):
    b = embeddings.shape[0]
    lab = target.astype(jnp.int32)
    out = pl.pallas_call(
        _triplet_loss_kernel,
        out_shape=jax.ShapeDtypeStruct((1, 1), jnp.float32),
    )(embeddings, lab.reshape(1, b))
    return out[0, 0]
